# baseline (speedup 1.0000x reference)
import jax
import jax.numpy as jnp
from jax import lax
from jax.experimental import pallas as pl
from jax.experimental.pallas import tpu as pltpu
from jax.experimental.pallas import tpu_sc as plsc

B = 16384
D = 64
V = 1_000_000
G = 3
R_TOT = G * B
UNIT = 512
NU = 1954
NUPAD = 1984
CNT_N = 1968
TAIL_UNIT = 1953
TAIL_COL = TAIL_UNIT * UNIT
NW = 32
EPW = B // NW
NBK = (TAIL_UNIT * 32 + 31) * 16 + 16
NCNT = CNT_N * 32
ACC_H = 224
DUMP = R_TOT * D
OUT_N = R_TOT * D + 64

_mesh = plsc.VectorSubcoreMesh(core_axis_name="c", subcore_axis_name="s")
_i32 = jnp.int32
_f32 = jnp.float32


def _k0_body(u_idx, p_idx, n_idx,
             bk_uu, bk_ur, bk_iu, bk_ir, cnts_u, cnts_i,
             idx_v, cur_u, cur_i, au_u, ar_u, ai_u, au_i, ar_i, ai_i,
             ci_v, sem):
    wid = lax.axis_index("s") * 2 + lax.axis_index("c")
    lanes = lax.iota(_i32, 16)
    oh0_i = jnp.where(lanes == 0, 1, 0).astype(_i32)
    oh0_b = lanes == 0
    zeros = jnp.zeros((16,), _i32)

    def zstep(k, _):
        cur_u[pl.ds(k * 16, 16)] = zeros
        cur_i[pl.ds(k * 16, 16)] = zeros
        return 0
    lax.fori_loop(0, NUPAD // 16, zstep, 0)

    for g, src in enumerate((u_idx, p_idx, n_idx)):
        pltpu.sync_copy(src.at[pl.ds(wid * EPW, EPW)], idx_v.at[pl.ds(0, EPW)])
        rbase = g * B + wid * EPW
        cur = cur_u if g == 0 else cur_i
        au, ar, ai = (au_u, ar_u, ai_u) if g == 0 else (au_i, ar_i, ai_i)
        pbase = 0 if g == 0 else (g - 1) * EPW

        def step(j, _, cur=cur, au=au, ar=ar, ai=ai, rbase=rbase, pbase=pbase):
            u = idx_v[pl.ds(j, 16)][0]
            unit = u // UNIT
            cw = cur[pl.ds(unit, 16)]
            c = cw[0]
            cur[pl.ds(unit, 16)] = cw + oh0_i
            c15 = jnp.minimum(c, 15)
            gs = (unit * 32 + wid) * 16 + c15
            p = pbase + j
            uw = au[pl.ds(p, 16)]
            au[pl.ds(p, 16)] = jnp.where(oh0_b, u, uw)
            rw = ar[pl.ds(p, 16)]
            ar[pl.ds(p, 16)] = jnp.where(oh0_b, rbase + j, rw)
            iw = ai[pl.ds(p, 16)]
            ai[pl.ds(p, 16)] = jnp.where(oh0_b, gs, iw)
            return 0
        lax.fori_loop(0, EPW, step, 0)

    def cwstep(k, _):
        ci_v[pl.ds(k * 16, 16)] = (lanes + k * 16) * 32 + wid
        return 0
    lax.fori_loop(0, CNT_N // 16, cwstep, 0)

    c1 = pltpu.async_copy(au_u.at[pl.ds(0, EPW)], bk_uu.at[ai_u.at[pl.ds(0, EPW)]], sem)
    c2 = pltpu.async_copy(ar_u.at[pl.ds(0, EPW)], bk_ur.at[ai_u.at[pl.ds(0, EPW)]], sem)
    c3 = pltpu.async_copy(au_i.at[pl.ds(0, 2 * EPW)], bk_iu.at[ai_i.at[pl.ds(0, 2 * EPW)]], sem)
    c4 = pltpu.async_copy(ar_i.at[pl.ds(0, 2 * EPW)], bk_ir.at[ai_i.at[pl.ds(0, 2 * EPW)]], sem)
    c5 = pltpu.async_copy(cur_u.at[pl.ds(0, CNT_N)], cnts_u.at[ci_v], sem)
    c6 = pltpu.async_copy(cur_i.at[pl.ds(0, CNT_N)], cnts_i.at[ci_v], sem)
    c1.wait(); c2.wait(); c3.wait(); c4.wait(); c5.wait(); c6.wait()


_k0 = pl.kernel(
    _k0_body,
    mesh=_mesh,
    out_type=(
        jax.ShapeDtypeStruct((NBK,), _i32),
        jax.ShapeDtypeStruct((NBK,), _i32),
        jax.ShapeDtypeStruct((NBK,), _i32),
        jax.ShapeDtypeStruct((NBK,), _i32),
        jax.ShapeDtypeStruct((NCNT,), _i32),
        jax.ShapeDtypeStruct((NCNT,), _i32),
    ),
    compiler_params=pltpu.CompilerParams(use_tc_tiling_on_sc=False),
    scratch_types=[
        pltpu.VMEM((EPW + 16,), _i32),
        pltpu.VMEM((NUPAD,), _i32),
        pltpu.VMEM((NUPAD,), _i32),
        pltpu.VMEM((EPW + 16,), _i32),
        pltpu.VMEM((EPW + 16,), _i32),
        pltpu.VMEM((EPW + 16,), _i32),
        pltpu.VMEM((2 * EPW + 16,), _i32),
        pltpu.VMEM((2 * EPW + 16,), _i32),
        pltpu.VMEM((2 * EPW + 16,), _i32),
        pltpu.VMEM((CNT_N,), _i32),
        pltpu.SemaphoreType.DMA,
    ],
)


def _k1_body(tT_u, tT_i, bk_uu, bk_ur, bk_iu, bk_ir, cnts_u, cnts_i,
             out_flat,
             slab, tslab, bu_v, br_v, cnt_v, acc_d, acc_i,
             sem_slab, sem_fl, sem_bkt):
    wid = lax.axis_index("s") * 2 + lax.axis_index("c")
    lanes = lax.iota(_i32, 16)

    def stage_buckets(bk_u_ref, bk_r_ref, cnts_ref, unit):
        b1 = pltpu.async_copy(bk_u_ref.at[pl.ds(unit * UNIT, UNIT)],
                              bu_v.at[pl.ds(0, UNIT)], sem_bkt)
        b2 = pltpu.async_copy(bk_r_ref.at[pl.ds(unit * UNIT, UNIT)],
                              br_v.at[pl.ds(0, UNIT)], sem_bkt)
        b3 = pltpu.async_copy(cnts_ref.at[pl.ds(unit * 32, 32)],
                              cnt_v.at[pl.ds(0, 32)], sem_bkt)
        b1.wait(); b2.wait(); b3.wait()

    def drain(n):
        def wk(k, _):
            pltpu.make_async_copy(acc_d.at[pl.ds(0, 512)],
                                  out_flat.at[acc_i.at[pl.ds(0, 512)]],
                                  sem_fl).wait()
            return 0
        lax.fori_loop(0, n, wk, 0)

    def process_set(slab_ref, col0, pnch):
        drain(pnch)

        def w2body(w2, hc):
            cnt = jnp.minimum(cnt_v[pl.ds(w2, 16)][0], 16)

            def hbody(h, hc2):
                q = w2 * 16 + h
                u = bu_v[pl.ds(q, 16)][0]
                r = br_v[pl.ds(q, 16)][0]
                ul = u - col0
                hc3 = jnp.minimum(hc2, ACC_H - 1)
                for m in range(4):
                    vals = plsc.load_gather(
                        slab_ref, [lanes + m * 16, jnp.full((16,), ul, _i32)])
                    acc_d[pl.ds(hc3 * 64 + m * 16, 16)] = vals
                    acc_i[pl.ds(hc3 * 64 + m * 16, 16)] = r * 64 + m * 16 + lanes
                return jnp.minimum(hc2 + 1, ACC_H)
            return lax.fori_loop(0, cnt, hbody, hc)

        hc = lax.fori_loop(0, 32, w2body, jnp.asarray(0, _i32))
        nch = (hc * 64 + 511) // 512

        def pw(p, _):
            acc_i[pl.ds(p * 16, 16)] = DUMP + lanes
            return 0
        lax.fori_loop(hc * 4, nch * 32, pw, 0)

        def fk(k, _):
            pltpu.async_copy(acc_d.at[pl.ds(k * 512, 512)],
                             out_flat.at[acc_i.at[pl.ds(k * 512, 512)]],
                             sem_fl)
            return 0
        lax.fori_loop(0, nch, fk, 0)
        return nch

    def unit_step(s, pbytes):
        unit = jnp.minimum(s * 32 + wid, TAIL_UNIT - 1)
        col0 = pl.multiple_of(unit * UNIT, 128)
        pltpu.sync_copy(tT_u.at[:, pl.ds(col0, UNIT)], slab)
        stage_buckets(bk_uu, bk_ur, cnts_u, unit)
        p1 = process_set(slab, col0, pbytes)
        pltpu.sync_copy(tT_i.at[:, pl.ds(col0, UNIT)], slab)
        stage_buckets(bk_iu, bk_ir, cnts_i, unit)
        return process_set(slab, col0, p1)

    pbytes = lax.fori_loop(0, 62, unit_step, jnp.asarray(0, _i32))

    @pl.when(wid == 1)
    def _tail():
        pltpu.sync_copy(tT_u.at[:, pl.ds(TAIL_COL, 64)], tslab)
        stage_buckets(bk_uu, bk_ur, cnts_u, TAIL_UNIT)
        p1 = process_set(tslab, TAIL_COL, pbytes)
        pltpu.sync_copy(tT_i.at[:, pl.ds(TAIL_COL, 64)], tslab)
        stage_buckets(bk_iu, bk_ir, cnts_i, TAIL_UNIT)
        p2 = process_set(tslab, TAIL_COL, p1)
        drain(p2)

    @pl.when(wid != 1)
    def _drain():
        drain(pbytes)


_k1 = pl.kernel(
    _k1_body,
    mesh=_mesh,
    out_type=jax.ShapeDtypeStruct((OUT_N,), _f32),
    compiler_params=pltpu.CompilerParams(needs_layout_passes=False),
    scratch_types=[
        pltpu.VMEM((D, UNIT), _f32),
        pltpu.VMEM((D, 64), _f32),
        pltpu.VMEM((UNIT + 16,), _i32),
        pltpu.VMEM((UNIT + 16,), _i32),
        pltpu.VMEM((48,), _i32),
        pltpu.VMEM((ACC_H * 64,), _f32),
        pltpu.VMEM((ACC_H * 64,), _i32),
        pltpu.SemaphoreType.DMA,
        pltpu.SemaphoreType.DMA,
        pltpu.SemaphoreType.DMA,
    ],
)


def kernel(batch_users, batch_pos_items, batch_neg_items, users_table, items_table):
    u = batch_users.astype(_i32)
    p = batch_pos_items.astype(_i32)
    n = batch_neg_items.astype(_i32)
    bk_uu, bk_ur, bk_iu, bk_ir, cnts_u, cnts_i = _k0(u, p, n)
    out_flat = _k1(users_table.T, items_table.T,
                   bk_uu, bk_ur, bk_iu, bk_ir, cnts_u, cnts_i)
    out3 = out_flat[:R_TOT * D].reshape(G, B, D)
    return (out3[0], out3[1], out3[2])


# speedup vs baseline: 29.3199x; 29.3199x over previous
import jax
import jax.numpy as jnp
from jax import lax
from jax.experimental import pallas as pl
from jax.experimental.pallas import tpu as pltpu
from jax.experimental.pallas import tpu_sc as plsc

B = 16384
D = 64
G = 3
R_TOT = G * B
UNIT = 512
TAIL_UNIT = 1953
TAIL_COL = TAIL_UNIT * UNIT
NU = 1954
NUPAD = 1984
NW = 32
EPW = B // NW
CAP = 6
CURN = NUPAD * 16
BPU = 32 * CAP * 16
NBKT = NU * BPU
NCNT = NU * 32 * 16
OUT_N = R_TOT * D

_mesh = plsc.VectorSubcoreMesh(core_axis_name="c", subcore_axis_name="s")
_i32 = jnp.int32
_f32 = jnp.float32


def _k0_body(u_idx, p_idx, n_idx,
             bk_uu, bk_ur, bk_iu, bk_ir, cnts_u, cnts_i,
             idx_v, cur_u, cur_i, accu, accr, acci, cidx2, sem):
    wid = lax.axis_index("s") * 2 + lax.axis_index("c")
    lanes = lax.iota(_i32, 16)
    zeros = jnp.zeros((16,), _i32)

    def zstep(k, _):
        for q in range(4):
            cur_u[pl.ds(k * 64 + q * 16, 16)] = zeros
            cur_i[pl.ds(k * 64 + q * 16, 16)] = zeros
        for q in range(4):
            u0 = k * 4 + q
            cidx2[pl.ds(u0 * 16, 16)] = (u0 * 32 + wid) * 16 + lanes
        return 0
    lax.fori_loop(0, CURN // 64, zstep, 0)

    for g, src in enumerate((u_idx, p_idx, n_idx)):
        pltpu.sync_copy(src.at[pl.ds(wid * EPW, EPW)], idx_v.at[pl.ds(0, EPW)])
        rbase = g * B + wid * EPW
        cur = cur_u if g == 0 else cur_i
        pbase = g * EPW

        def kstep(k, _, cur=cur, rbase=rbase, pbase=pbase):
            u_vec = idx_v[pl.ds(k * 16, 16)]
            unit_vec = u_vec // UNIT
            cidx = unit_vec * 16 + lanes
            c_vec = plsc.load_gather(cur, [cidx])
            plsc.store_scatter(cur, [cidx], c_vec + 1)
            c5 = jnp.minimum(c_vec, CAP - 1)
            gslot = ((unit_vec * 32 + wid) * CAP + c5) * 16 + lanes
            accu[pl.ds(pbase + k * 16, 16)] = u_vec
            accr[pl.ds(pbase + k * 16, 16)] = rbase + k * 16 + lanes
            acci[pl.ds(pbase + k * 16, 16)] = gslot
            return 0
        lax.fori_loop(0, EPW // 16, kstep, 0)

    c1 = pltpu.async_copy(accu.at[pl.ds(0, EPW)], bk_uu.at[acci.at[pl.ds(0, EPW)]], sem)
    c2 = pltpu.async_copy(accr.at[pl.ds(0, EPW)], bk_ur.at[acci.at[pl.ds(0, EPW)]], sem)
    c3 = pltpu.async_copy(accu.at[pl.ds(EPW, 2 * EPW)], bk_iu.at[acci.at[pl.ds(EPW, 2 * EPW)]], sem)
    c4 = pltpu.async_copy(accr.at[pl.ds(EPW, 2 * EPW)], bk_ir.at[acci.at[pl.ds(EPW, 2 * EPW)]], sem)
    c5 = pltpu.async_copy(cur_u.at[pl.ds(0, NU * 16)], cnts_u.at[cidx2.at[pl.ds(0, NU * 16)]], sem)
    c6 = pltpu.async_copy(cur_i.at[pl.ds(0, NU * 16)], cnts_i.at[cidx2.at[pl.ds(0, NU * 16)]], sem)
    c1.wait(); c2.wait(); c3.wait(); c4.wait(); c5.wait(); c6.wait()


_k0 = pl.kernel(
    _k0_body,
    mesh=_mesh,
    out_type=(
        jax.ShapeDtypeStruct((NBKT,), _i32),
        jax.ShapeDtypeStruct((NBKT,), _i32),
        jax.ShapeDtypeStruct((NBKT,), _i32),
        jax.ShapeDtypeStruct((NBKT,), _i32),
        jax.ShapeDtypeStruct((NCNT,), _i32),
        jax.ShapeDtypeStruct((NCNT,), _i32),
    ),
    compiler_params=pltpu.CompilerParams(use_tc_tiling_on_sc=False,
                                         needs_layout_passes=False),
    scratch_types=[
        pltpu.VMEM((EPW + 16,), _i32),
        pltpu.VMEM((CURN,), _i32),
        pltpu.VMEM((CURN,), _i32),
        pltpu.VMEM((G * EPW,), _i32),
        pltpu.VMEM((G * EPW,), _i32),
        pltpu.VMEM((G * EPW,), _i32),
        pltpu.VMEM((CURN,), _i32),
        pltpu.SemaphoreType.DMA,
    ],
)


def _k1_body(tT_u, tT_i, bk_uu, bk_ur, bk_iu, bk_ir, cnts_u, cnts_i,
             out_flat,
             slab, tslab, bku_v, bkr_v, cnt_v, rowring,
             sem_slab, sem_bkt, sem_out):
    wid = lax.axis_index("s") * 2 + lax.axis_index("c")
    lanes = lax.iota(_i32, 16)

    def fill_slab(tT, col0):
        for R in range(8):
            pltpu.async_copy(tT.at[pl.ds(R * 8, 8), pl.ds(col0, UNIT)],
                             slab.at[pl.ds(R * 8, 8), :], sem_slab)
        for R in range(8):
            pltpu.make_async_copy(tT.at[pl.ds(R * 8, 8), pl.ds(col0, UNIT)],
                                  slab.at[pl.ds(R * 8, 8), :], sem_slab).wait()

    def fill_tslab(tT):
        for R in range(8):
            pltpu.async_copy(tT.at[pl.ds(R * 8, 8), pl.ds(TAIL_COL, 64)],
                             tslab.at[pl.ds(R * 8, 8), :], sem_slab)
        for R in range(8):
            pltpu.make_async_copy(tT.at[pl.ds(R * 8, 8), pl.ds(TAIL_COL, 64)],
                                  tslab.at[pl.ds(R * 8, 8), :], sem_slab).wait()

    def stage_buckets(bk_u_ref, bk_r_ref, cnts_ref, unit):
        b1 = pltpu.async_copy(bk_u_ref.at[pl.ds(unit * BPU, BPU)],
                              bku_v.at[pl.ds(0, BPU)], sem_bkt)
        b2 = pltpu.async_copy(bk_r_ref.at[pl.ds(unit * BPU, BPU)],
                              bkr_v.at[pl.ds(0, BPU)], sem_bkt)
        b3 = pltpu.async_copy(cnts_ref.at[pl.ds(unit * 512, 512)],
                              cnt_v.at[pl.ds(0, 512)], sem_bkt)
        b1.wait(); b2.wait(); b3.wait()

    def drain_out(n):
        def wk(t, _):
            pltpu.make_async_copy(rowring.at[pl.ds(0, 64)],
                                  out_flat.at[pl.ds(0, 64)], sem_out).wait()
            return 0
        lax.fori_loop(0, n, wk, 0)

    HRING = CAP * 16 * 64

    def process_set(slab_ref, col0, pend):
        def w2body(w2, st):
            p_even, p_odd = st
            cvec = cnt_v[pl.ds(w2 * 16, 16)]
            tot = lax.reduce_sum(jnp.minimum(cvec, CAP), (0,))
            half = w2 % 2
            base = half * HRING
            drain_out(jnp.where(half == 0, p_even, p_odd))

            @pl.when(tot > 0)
            def _scan():
                for c in range(CAP):
                    mask = cvec > c
                    nh = plsc.all_reduce_population_count(mask)[0]
                    boff = w2 * (CAP * 16) + c * 16

                    def hbody(t, m):
                        L = plsc.all_reduce_ffs(m)[0]
                        u_h = plsc.load_gather(bku_v, [jnp.full((16,), boff + L, _i32)])[0]
                        r_h = plsc.load_gather(bkr_v, [jnp.full((16,), boff + L, _i32)])[0]
                        ul = u_h - col0
                        so = base + (c * 16 + L) * 64
                        for m4 in range(4):
                            vals = plsc.load_gather(
                                slab_ref, [lanes + m4 * 16, jnp.full((16,), ul, _i32)])
                            rowring[pl.ds(so + m4 * 16, 16)] = vals
                        ro = pl.multiple_of(r_h * 64, 64)
                        so8 = pl.multiple_of(so, 64)
                        pltpu.async_copy(rowring.at[pl.ds(so8, 64)],
                                         out_flat.at[pl.ds(ro, 64)], sem_out)
                        return m & (lanes != L)
                    lax.fori_loop(0, nh, hbody, mask)
            p_even2 = jnp.where(half == 0, tot, p_even)
            p_odd2 = jnp.where(half == 0, p_odd, tot)
            return (p_even2, p_odd2)
        return lax.fori_loop(0, 32, w2body, pend)

    zz = jnp.asarray(0, _i32)

    def unit_step(s, _):
        unit = jnp.minimum(s * 32 + wid, TAIL_UNIT - 1)
        col0 = pl.multiple_of(unit * UNIT, 128)
        fill_slab(tT_u, col0)
        stage_buckets(bk_uu, bk_ur, cnts_u, unit)
        p = process_set(slab, col0, (zz, zz))
        drain_out(p[0] + p[1])
        fill_slab(tT_i, col0)
        stage_buckets(bk_iu, bk_ir, cnts_i, unit)
        p = process_set(slab, col0, (zz, zz))
        drain_out(p[0] + p[1])
        return 0

    lax.fori_loop(0, 62, unit_step, 0)

    @pl.when(wid == 1)
    def _tail():
        fill_tslab(tT_u)
        stage_buckets(bk_uu, bk_ur, cnts_u, TAIL_UNIT)
        p = process_set(tslab, TAIL_COL, (zz, zz))
        drain_out(p[0] + p[1])
        fill_tslab(tT_i)
        stage_buckets(bk_iu, bk_ir, cnts_i, TAIL_UNIT)
        p = process_set(tslab, TAIL_COL, (zz, zz))
        drain_out(p[0] + p[1])


_k1 = pl.kernel(
    _k1_body,
    mesh=_mesh,
    out_type=jax.ShapeDtypeStruct((OUT_N,), _f32),
    compiler_params=pltpu.CompilerParams(needs_layout_passes=False),
    scratch_types=[
        pltpu.VMEM((D, UNIT), _f32),
        pltpu.VMEM((D, 64), _f32),
        pltpu.VMEM((BPU,), _i32),
        pltpu.VMEM((BPU,), _i32),
        pltpu.VMEM((512 + 16,), _i32),
        pltpu.VMEM((2 * CAP * 16 * 64,), _f32),
        pltpu.SemaphoreType.DMA,
        pltpu.SemaphoreType.DMA,
        pltpu.SemaphoreType.DMA,
    ],
)


def kernel(batch_users, batch_pos_items, batch_neg_items, users_table, items_table):
    u = batch_users.astype(_i32)
    p = batch_pos_items.astype(_i32)
    n = batch_neg_items.astype(_i32)
    bk_uu, bk_ur, bk_iu, bk_ir, cnts_u, cnts_i = _k0(u, p, n)
    out_flat = _k1(users_table.T, items_table.T,
                   bk_uu, bk_ur, bk_iu, bk_ir, cnts_u, cnts_i)
    out3 = out_flat.reshape(G, B, D)
    return (out3[0], out3[1], out3[2])


# speedup vs baseline: 151.8716x; 5.1798x over previous
import jax
import jax.numpy as jnp
from jax import lax
from jax.experimental import pallas as pl
from jax.experimental.pallas import tpu as pltpu
from jax.experimental.pallas import tpu_sc as plsc

B = 16384
D = 64
G = 3
R_TOT = G * B
UNIT = 512
TAIL_UNIT = 1953
TAIL_COL = TAIL_UNIT * UNIT
NU = 1954
NUPAD = 1984
NW = 32
EPW = B // NW
CAP = 6
CURN = NUPAD * 16
BPU = 32 * CAP * 16
NBKT = NU * BPU
CROW = NU * 16
CROWP = CROW + 16
UPW = 62
OUT_N = R_TOT * D

_mesh = plsc.VectorSubcoreMesh(core_axis_name="c", subcore_axis_name="s")
_i32 = jnp.int32
_f32 = jnp.float32


def _k0_body(u_idx, p_idx, n_idx,
             bk_uu, bk_ur, bk_iu, bk_ir, cnts_u, cnts_i,
             idx_v, cur_u, cur_i, accu, accr, acci, sem):
    wid = lax.axis_index("s") * 2 + lax.axis_index("c")
    lanes = lax.iota(_i32, 16)
    zeros = jnp.zeros((16,), _i32)

    def zstep(k, _):
        for q in range(4):
            cur_u[pl.ds(k * 64 + q * 16, 16)] = zeros
            cur_i[pl.ds(k * 64 + q * 16, 16)] = zeros
        return 0
    lax.fori_loop(0, CURN // 64, zstep, 0)

    for g, src in enumerate((u_idx, p_idx, n_idx)):
        pltpu.sync_copy(src.at[pl.ds(wid * EPW, EPW)], idx_v.at[pl.ds(0, EPW)])
        rbase = g * B + wid * EPW
        cur = cur_u if g == 0 else cur_i
        pbase = g * EPW

        def kstep(k, _, cur=cur, rbase=rbase, pbase=pbase):
            u_vec = idx_v[pl.ds(k * 16, 16)]
            unit_vec = u_vec // UNIT
            cidx = unit_vec * 16 + lanes
            c_vec = plsc.load_gather(cur, [cidx])
            plsc.store_scatter(cur, [cidx], c_vec + 1)
            c5 = jnp.minimum(c_vec, CAP - 1)
            gslot = ((unit_vec * 32 + wid) * CAP + c5) * 16 + lanes
            accu[pl.ds(pbase + k * 16, 16)] = u_vec
            accr[pl.ds(pbase + k * 16, 16)] = rbase + k * 16 + lanes
            acci[pl.ds(pbase + k * 16, 16)] = gslot
            return 0
        lax.fori_loop(0, EPW // 16, kstep, 0)

    c1 = pltpu.async_copy(accu.at[pl.ds(0, EPW)], bk_uu.at[acci.at[pl.ds(0, EPW)]], sem)
    c2 = pltpu.async_copy(accr.at[pl.ds(0, EPW)], bk_ur.at[acci.at[pl.ds(0, EPW)]], sem)
    c3 = pltpu.async_copy(accu.at[pl.ds(EPW, 2 * EPW)], bk_iu.at[acci.at[pl.ds(EPW, 2 * EPW)]], sem)
    c4 = pltpu.async_copy(accr.at[pl.ds(EPW, 2 * EPW)], bk_ir.at[acci.at[pl.ds(EPW, 2 * EPW)]], sem)
    wb = wid * CROWP
    c5 = pltpu.async_copy(cur_u.at[pl.ds(0, CROW)], cnts_u.at[pl.ds(wb, CROW)], sem)
    c6 = pltpu.async_copy(cur_i.at[pl.ds(0, CROW)], cnts_i.at[pl.ds(wb, CROW)], sem)
    c1.wait(); c2.wait(); c3.wait(); c4.wait(); c5.wait(); c6.wait()


_k0 = pl.kernel(
    _k0_body,
    mesh=_mesh,
    out_type=(
        jax.ShapeDtypeStruct((NBKT,), _i32),
        jax.ShapeDtypeStruct((NBKT,), _i32),
        jax.ShapeDtypeStruct((NBKT,), _i32),
        jax.ShapeDtypeStruct((NBKT,), _i32),
        jax.ShapeDtypeStruct((NW * CROWP,), _i32),
        jax.ShapeDtypeStruct((NW * CROWP,), _i32),
    ),
    compiler_params=pltpu.CompilerParams(use_tc_tiling_on_sc=False,
                                         needs_layout_passes=False),
    scratch_types=[
        pltpu.VMEM((EPW + 16,), _i32),
        pltpu.VMEM((CURN,), _i32),
        pltpu.VMEM((CURN,), _i32),
        pltpu.VMEM((G * EPW,), _i32),
        pltpu.VMEM((G * EPW,), _i32),
        pltpu.VMEM((G * EPW,), _i32),
        pltpu.SemaphoreType.DMA,
    ],
)


def _k1_body(tT_u, tT_i, bk_uu, bk_ur, bk_iu, bk_ir, cnts_u, cnts_i,
             out_flat,
             slab, tslab, bku_v, bkr_v, cntu_v, cnti_v, rowring,
             sem_slab, sem_bkt, sem_out):
    wid = lax.axis_index("s") * 2 + lax.axis_index("c")
    lanes = lax.iota(_i32, 16)
    wbase = wid * 61 + jnp.minimum(wid, 2)
    mylast = jnp.where(wid < 2, 61, jnp.where(wid == 31, 59, 60))

    for w2 in range(32):
        pltpu.async_copy(cnts_u.at[pl.ds(w2 * CROWP + wbase * 16, UPW * 16)],
                         cntu_v.at[pl.ds(w2 * (UPW * 16), UPW * 16)], sem_bkt)
        pltpu.async_copy(cnts_i.at[pl.ds(w2 * CROWP + wbase * 16, UPW * 16)],
                         cnti_v.at[pl.ds(w2 * (UPW * 16), UPW * 16)], sem_bkt)
    for w2 in range(32):
        pltpu.make_async_copy(cnts_u.at[pl.ds(w2 * CROWP + wbase * 16, UPW * 16)],
                              cntu_v.at[pl.ds(w2 * (UPW * 16), UPW * 16)], sem_bkt).wait()
        pltpu.make_async_copy(cnts_i.at[pl.ds(w2 * CROWP + wbase * 16, UPW * 16)],
                              cnti_v.at[pl.ds(w2 * (UPW * 16), UPW * 16)], sem_bkt).wait()

    def fill_slab(tT, col0):
        for R in range(8):
            pltpu.async_copy(tT.at[pl.ds(R * 8, 8), pl.ds(col0, UNIT)],
                             slab.at[pl.ds(R * 8, 8), :], sem_slab)
        for R in range(8):
            pltpu.make_async_copy(tT.at[pl.ds(R * 8, 8), pl.ds(col0, UNIT)],
                                  slab.at[pl.ds(R * 8, 8), :], sem_slab).wait()

    def fill_tslab(tT):
        for R in range(8):
            pltpu.async_copy(tT.at[pl.ds(R * 8, 8), pl.ds(TAIL_COL, 64)],
                             tslab.at[pl.ds(R * 8, 8), :], sem_slab)
        for R in range(8):
            pltpu.make_async_copy(tT.at[pl.ds(R * 8, 8), pl.ds(TAIL_COL, 64)],
                                  tslab.at[pl.ds(R * 8, 8), :], sem_slab).wait()

    def stage_buckets(bk_u_ref, bk_r_ref, unit):
        b1 = pltpu.async_copy(bk_u_ref.at[pl.ds(unit * BPU, BPU)],
                              bku_v.at[pl.ds(0, BPU)], sem_bkt)
        b2 = pltpu.async_copy(bk_r_ref.at[pl.ds(unit * BPU, BPU)],
                              bkr_v.at[pl.ds(0, BPU)], sem_bkt)
        b1.wait(); b2.wait()

    def drain_out(n):
        def wk(t, _):
            pltpu.make_async_copy(rowring.at[pl.ds(0, 64)],
                                  out_flat.at[pl.ds(0, 64)], sem_out).wait()
            return 0
        lax.fori_loop(0, n, wk, 0)

    HRING = CAP * 16 * 64

    def process_set(slab_ref, col0, cnt_ref, s_l, pend):
        def w2body(w2, st):
            p_even, p_odd = st
            cvec = cnt_ref[pl.ds(w2 * (UPW * 16) + s_l * 16, 16)]
            tot = lax.reduce_sum(jnp.minimum(cvec, CAP), (0,))
            half = w2 % 2
            base = half * HRING
            drain_out(jnp.where(half == 0, p_even, p_odd))

            @pl.when(tot > 0)
            def _scan():
                for c in range(CAP):
                    mask = cvec > c
                    nh = plsc.all_reduce_population_count(mask)[0]
                    boff = w2 * (CAP * 16) + c * 16

                    def hbody(t, m):
                        L = plsc.all_reduce_ffs(m)[0]
                        u_h = plsc.load_gather(bku_v, [jnp.full((16,), boff + L, _i32)])[0]
                        r_h = plsc.load_gather(bkr_v, [jnp.full((16,), boff + L, _i32)])[0]
                        ul = u_h - col0
                        so = base + (c * 16 + L) * 64
                        for m4 in range(4):
                            vals = plsc.load_gather(
                                slab_ref, [lanes + m4 * 16, jnp.full((16,), ul, _i32)])
                            rowring[pl.ds(so + m4 * 16, 16)] = vals
                        ro = pl.multiple_of(r_h * 64, 64)
                        so8 = pl.multiple_of(so, 64)
                        pltpu.async_copy(rowring.at[pl.ds(so8, 64)],
                                         out_flat.at[pl.ds(ro, 64)], sem_out)
                        return m & (lanes != L)
                    lax.fori_loop(0, nh, hbody, mask)
            p_even2 = jnp.where(half == 0, tot, p_even)
            p_odd2 = jnp.where(half == 0, p_odd, tot)
            return (p_even2, p_odd2)
        return lax.fori_loop(0, 32, w2body, pend)

    zz = jnp.asarray(0, _i32)

    def unit_step(s, _):
        s_l = jnp.minimum(s, mylast)
        unit = wbase + s_l
        col0 = pl.multiple_of(unit * UNIT, 128)
        fill_slab(tT_u, col0)
        stage_buckets(bk_uu, bk_ur, unit)
        p = process_set(slab, col0, cntu_v, s_l, (zz, zz))
        drain_out(p[0] + p[1])
        fill_slab(tT_i, col0)
        stage_buckets(bk_iu, bk_ir, unit)
        p = process_set(slab, col0, cnti_v, s_l, (zz, zz))
        drain_out(p[0] + p[1])
        return 0

    lax.fori_loop(0, 62, unit_step, 0)

    @pl.when(wid == 31)
    def _tail():
        fill_tslab(tT_u)
        stage_buckets(bk_uu, bk_ur, TAIL_UNIT)
        p = process_set(tslab, TAIL_COL, cntu_v, 60, (zz, zz))
        drain_out(p[0] + p[1])
        fill_tslab(tT_i)
        stage_buckets(bk_iu, bk_ir, TAIL_UNIT)
        p = process_set(tslab, TAIL_COL, cnti_v, 60, (zz, zz))
        drain_out(p[0] + p[1])


_k1 = pl.kernel(
    _k1_body,
    mesh=_mesh,
    out_type=jax.ShapeDtypeStruct((OUT_N,), _f32),
    compiler_params=pltpu.CompilerParams(needs_layout_passes=False),
    scratch_types=[
        pltpu.VMEM((D, UNIT), _f32),
        pltpu.VMEM((D, 64), _f32),
        pltpu.VMEM((BPU,), _i32),
        pltpu.VMEM((BPU,), _i32),
        pltpu.VMEM((32 * UPW * 16,), _i32),
        pltpu.VMEM((32 * UPW * 16,), _i32),
        pltpu.VMEM((2 * CAP * 16 * 64,), _f32),
        pltpu.SemaphoreType.DMA,
        pltpu.SemaphoreType.DMA,
        pltpu.SemaphoreType.DMA,
    ],
)


def kernel(batch_users, batch_pos_items, batch_neg_items, users_table, items_table):
    u = batch_users.astype(_i32)
    p = batch_pos_items.astype(_i32)
    n = batch_neg_items.astype(_i32)
    bk_uu, bk_ur, bk_iu, bk_ir, cnts_u, cnts_i = _k0(u, p, n)
    out_flat = _k1(users_table.T, items_table.T,
                   bk_uu, bk_ur, bk_iu, bk_ir, cnts_u, cnts_i)
    out3 = out_flat.reshape(G, B, D)
    return (out3[0], out3[1], out3[2])


# speedup vs baseline: 185.7586x; 1.2231x over previous
import jax
import jax.numpy as jnp
from jax import lax
from jax.experimental import pallas as pl
from jax.experimental.pallas import tpu as pltpu
from jax.experimental.pallas import tpu_sc as plsc

B = 16384
D = 64
G = 3
R_TOT = G * B
UNIT = 512
TAIL_UNIT = 1953
TAIL_COL = TAIL_UNIT * UNIT
NU = 1954
NUPAD = 1984
NW = 32
EPW = B // NW
CAP = 6
CURN = NUPAD * 16
BPU = 32 * CAP * 16
NBKT = NU * BPU
CROW = NU * 16
CROWP = CROW + 16
UPW = 62
OUT_N = R_TOT * D

_mesh = plsc.VectorSubcoreMesh(core_axis_name="c", subcore_axis_name="s")
_i32 = jnp.int32
_f32 = jnp.float32


def _k0_body(u_idx, p_idx, n_idx,
             bk_uu, bk_ur, bk_iu, bk_ir, cnts_u, cnts_i,
             idx_v, cur_u, cur_i, accu, accr, acci, sem):
    wid = lax.axis_index("s") * 2 + lax.axis_index("c")
    lanes = lax.iota(_i32, 16)
    zeros = jnp.zeros((16,), _i32)

    def zstep(k, _):
        for q in range(4):
            cur_u[pl.ds(k * 64 + q * 16, 16)] = zeros
            cur_i[pl.ds(k * 64 + q * 16, 16)] = zeros
        return 0
    lax.fori_loop(0, CURN // 64, zstep, 0)

    for g, src in enumerate((u_idx, p_idx, n_idx)):
        pltpu.sync_copy(src.at[pl.ds(wid * EPW, EPW)], idx_v.at[pl.ds(0, EPW)])
        rbase = g * B + wid * EPW
        cur = cur_u if g == 0 else cur_i
        pbase = g * EPW

        def kstep(k, _, cur=cur, rbase=rbase, pbase=pbase):
            u_vec = idx_v[pl.ds(k * 16, 16)]
            unit_vec = u_vec // UNIT
            cidx = unit_vec * 16 + lanes
            c_vec = plsc.load_gather(cur, [cidx])
            plsc.store_scatter(cur, [cidx], c_vec + 1)
            c5 = jnp.minimum(c_vec, CAP - 1)
            gslot = ((unit_vec * 32 + wid) * CAP + c5) * 16 + lanes
            accu[pl.ds(pbase + k * 16, 16)] = u_vec
            accr[pl.ds(pbase + k * 16, 16)] = rbase + k * 16 + lanes
            acci[pl.ds(pbase + k * 16, 16)] = gslot
            return 0
        lax.fori_loop(0, EPW // 16, kstep, 0)

    c1 = pltpu.async_copy(accu.at[pl.ds(0, EPW)], bk_uu.at[acci.at[pl.ds(0, EPW)]], sem)
    c2 = pltpu.async_copy(accr.at[pl.ds(0, EPW)], bk_ur.at[acci.at[pl.ds(0, EPW)]], sem)
    c3 = pltpu.async_copy(accu.at[pl.ds(EPW, 2 * EPW)], bk_iu.at[acci.at[pl.ds(EPW, 2 * EPW)]], sem)
    c4 = pltpu.async_copy(accr.at[pl.ds(EPW, 2 * EPW)], bk_ir.at[acci.at[pl.ds(EPW, 2 * EPW)]], sem)
    wb = wid * CROWP
    c5 = pltpu.async_copy(cur_u.at[pl.ds(0, CROW)], cnts_u.at[pl.ds(wb, CROW)], sem)
    c6 = pltpu.async_copy(cur_i.at[pl.ds(0, CROW)], cnts_i.at[pl.ds(wb, CROW)], sem)
    c1.wait(); c2.wait(); c3.wait(); c4.wait(); c5.wait(); c6.wait()


_k0 = pl.kernel(
    _k0_body,
    mesh=_mesh,
    out_type=(
        jax.ShapeDtypeStruct((NBKT,), _i32),
        jax.ShapeDtypeStruct((NBKT,), _i32),
        jax.ShapeDtypeStruct((NBKT,), _i32),
        jax.ShapeDtypeStruct((NBKT,), _i32),
        jax.ShapeDtypeStruct((NW * CROWP,), _i32),
        jax.ShapeDtypeStruct((NW * CROWP,), _i32),
    ),
    compiler_params=pltpu.CompilerParams(use_tc_tiling_on_sc=False,
                                         needs_layout_passes=False),
    scratch_types=[
        pltpu.VMEM((EPW + 16,), _i32),
        pltpu.VMEM((CURN,), _i32),
        pltpu.VMEM((CURN,), _i32),
        pltpu.VMEM((G * EPW,), _i32),
        pltpu.VMEM((G * EPW,), _i32),
        pltpu.VMEM((G * EPW,), _i32),
        pltpu.SemaphoreType.DMA,
    ],
)


def _k1_body(tT_u, tT_i, bk_uu, bk_ur, bk_iu, bk_ir, cnts_u, cnts_i,
             out_flat,
             slab, slab_b, tslab, bku_v, bkr_v, cntu_v, cnti_v, rowring,
             sem_slab, sem_slab_b, sem_bkt, sem_out):
    wid = lax.axis_index("s") * 2 + lax.axis_index("c")
    lanes = lax.iota(_i32, 16)
    wbase = wid * 61 + jnp.minimum(wid, 2)
    mylast = jnp.where(wid < 2, 61, jnp.where(wid == 31, 59, 60))
    CB = 31 * 16

    def stage_counts(block):
        for w2 in range(32):
            pltpu.async_copy(cnts_u.at[pl.ds(w2 * CROWP + wbase * 16 + block * CB, CB)],
                             cntu_v.at[pl.ds(w2 * CB, CB)], sem_bkt)
            pltpu.async_copy(cnts_i.at[pl.ds(w2 * CROWP + wbase * 16 + block * CB, CB)],
                             cnti_v.at[pl.ds(w2 * CB, CB)], sem_bkt)
        for w2 in range(32):
            pltpu.make_async_copy(cnts_u.at[pl.ds(w2 * CROWP + wbase * 16 + block * CB, CB)],
                                  cntu_v.at[pl.ds(w2 * CB, CB)], sem_bkt).wait()
            pltpu.make_async_copy(cnts_i.at[pl.ds(w2 * CROWP + wbase * 16 + block * CB, CB)],
                                  cnti_v.at[pl.ds(w2 * CB, CB)], sem_bkt).wait()

    stage_counts(jnp.asarray(0, _i32))

    def issue_fill(buf, sem, tT, col0):
        for R in range(8):
            pltpu.async_copy(tT.at[pl.ds(R * 8, 8), pl.ds(col0, UNIT)],
                             buf.at[pl.ds(R * 8, 8), :], sem)

    def wait_fill(buf, sem, tT, col0):
        for R in range(8):
            pltpu.make_async_copy(tT.at[pl.ds(R * 8, 8), pl.ds(col0, UNIT)],
                                  buf.at[pl.ds(R * 8, 8), :], sem).wait()

    def fill_tslab(tT):
        for R in range(8):
            pltpu.async_copy(tT.at[pl.ds(R * 8, 8), pl.ds(TAIL_COL, 64)],
                             tslab.at[pl.ds(R * 8, 8), :], sem_slab)
        for R in range(8):
            pltpu.make_async_copy(tT.at[pl.ds(R * 8, 8), pl.ds(TAIL_COL, 64)],
                                  tslab.at[pl.ds(R * 8, 8), :], sem_slab).wait()

    def stage_buckets(bk_u_ref, bk_r_ref, unit):
        b1 = pltpu.async_copy(bk_u_ref.at[pl.ds(unit * BPU, BPU)],
                              bku_v.at[pl.ds(0, BPU)], sem_bkt)
        b2 = pltpu.async_copy(bk_r_ref.at[pl.ds(unit * BPU, BPU)],
                              bkr_v.at[pl.ds(0, BPU)], sem_bkt)
        b1.wait(); b2.wait()

    def drain_out(n):
        def wk(t, _):
            pltpu.make_async_copy(rowring.at[pl.ds(0, 64)],
                                  out_flat.at[pl.ds(0, 64)], sem_out).wait()
            return 0
        lax.fori_loop(0, n, wk, 0)

    HRING = CAP * 16 * 64

    def process_set(slab_ref, col0, cnt_ref, s_b, pend):
        def w2body(w2, st):
            p_even, p_odd = st
            cvec = cnt_ref[pl.ds(w2 * CB + s_b * 16, 16)]
            tot = lax.reduce_sum(jnp.minimum(cvec, CAP), (0,))
            half = w2 % 2
            base = half * HRING
            drain_out(jnp.where(half == 0, p_even, p_odd))

            @pl.when(tot > 0)
            def _scan():
                for c in range(CAP):
                    mask = cvec > c
                    nh = plsc.all_reduce_population_count(mask)[0]
                    boff = w2 * (CAP * 16) + c * 16

                    def hbody(t, m):
                        L = plsc.all_reduce_ffs(m)[0]
                        u_h = plsc.load_gather(bku_v, [jnp.full((16,), boff + L, _i32)])[0]
                        r_h = plsc.load_gather(bkr_v, [jnp.full((16,), boff + L, _i32)])[0]
                        ul = u_h - col0
                        so = base + (c * 16 + L) * 64
                        for m4 in range(4):
                            vals = plsc.load_gather(
                                slab_ref, [lanes + m4 * 16, jnp.full((16,), ul, _i32)])
                            rowring[pl.ds(so + m4 * 16, 16)] = vals
                        ro = pl.multiple_of(r_h * 64, 64)
                        so8 = pl.multiple_of(so, 64)
                        pltpu.async_copy(rowring.at[pl.ds(so8, 64)],
                                         out_flat.at[pl.ds(ro, 64)], sem_out)
                        return m & (lanes != L)
                    lax.fori_loop(0, nh, hbody, mask)
            p_even2 = jnp.where(half == 0, tot, p_even)
            p_odd2 = jnp.where(half == 0, p_odd, tot)
            return (p_even2, p_odd2)
        return lax.fori_loop(0, 32, w2body, pend)

    zz = jnp.asarray(0, _i32)

    def colof(s):
        return pl.multiple_of((wbase + jnp.minimum(s, mylast)) * UNIT, 128)

    issue_fill(slab, sem_slab, tT_u, colof(0))

    def unit_step(s, _):
        s_l = jnp.minimum(s, mylast)
        unit = wbase + s_l
        col0 = colof(s)

        @pl.when(s == 31)
        def _restage():
            stage_counts(jnp.asarray(1, _i32))
        s_b = jnp.where(s_l >= 31, s_l - 31, s_l)

        wait_fill(slab, sem_slab, tT_u, col0)
        issue_fill(slab_b, sem_slab_b, tT_i, col0)
        stage_buckets(bk_uu, bk_ur, unit)
        p = process_set(slab, col0, cntu_v, s_b, (zz, zz))
        drain_out(p[0] + p[1])
        wait_fill(slab_b, sem_slab_b, tT_i, col0)
        issue_fill(slab, sem_slab, tT_u, colof(s + 1))
        stage_buckets(bk_iu, bk_ir, unit)
        p = process_set(slab_b, col0, cnti_v, s_b, (zz, zz))
        drain_out(p[0] + p[1])
        return 0

    lax.fori_loop(0, 62, unit_step, 0)
    wait_fill(slab, sem_slab, tT_u, colof(62))

    @pl.when(wid == 31)
    def _tail():
        fill_tslab(tT_u)
        stage_buckets(bk_uu, bk_ur, TAIL_UNIT)
        p = process_set(tslab, TAIL_COL, cntu_v, 29, (zz, zz))
        drain_out(p[0] + p[1])
        fill_tslab(tT_i)
        stage_buckets(bk_iu, bk_ir, TAIL_UNIT)
        p = process_set(tslab, TAIL_COL, cnti_v, 29, (zz, zz))
        drain_out(p[0] + p[1])


_k1 = pl.kernel(
    _k1_body,
    mesh=_mesh,
    out_type=jax.ShapeDtypeStruct((OUT_N,), _f32),
    compiler_params=pltpu.CompilerParams(needs_layout_passes=False),
    scratch_types=[
        pltpu.VMEM((D, UNIT), _f32),
        pltpu.VMEM((D, UNIT), _f32),
        pltpu.VMEM((D, 64), _f32),
        pltpu.VMEM((BPU,), _i32),
        pltpu.VMEM((BPU,), _i32),
        pltpu.VMEM((32 * 31 * 16,), _i32),
        pltpu.VMEM((32 * 31 * 16,), _i32),
        pltpu.VMEM((2 * CAP * 16 * 64,), _f32),
        pltpu.SemaphoreType.DMA,
        pltpu.SemaphoreType.DMA,
        pltpu.SemaphoreType.DMA,
        pltpu.SemaphoreType.DMA,
    ],
)


def kernel(batch_users, batch_pos_items, batch_neg_items, users_table, items_table):
    u = batch_users.astype(_i32)
    p = batch_pos_items.astype(_i32)
    n = batch_neg_items.astype(_i32)
    bk_uu, bk_ur, bk_iu, bk_ir, cnts_u, cnts_i = _k0(u, p, n)
    out_flat = _k1(users_table.T, items_table.T,
                   bk_uu, bk_ur, bk_iu, bk_ir, cnts_u, cnts_i)
    out3 = out_flat.reshape(G, B, D)
    return (out3[0], out3[1], out3[2])
